# baseline (speedup 1.0000x reference)
import functools

import jax
import jax.numpy as jnp
from jax import lax
from jax.experimental import pallas as pl
from jax.experimental.pallas import tpu as pltpu
from jax.experimental.pallas import tpu_sc as plsc

TOPK = 16
EPS = 1e-6
_NTIES = 3


def _sc_top16(corr_t):
    BC, L = corr_t.shape
    info = plsc.get_sparse_core_info()
    nw = info.num_cores * info.num_subcores
    rows_per_tile = BC // nw
    nchunks = L // 16

    @functools.partial(
        pl.kernel,
        mesh=plsc.VectorSubcoreMesh(core_axis_name="c", subcore_axis_name="s"),
        out_type=jax.ShapeDtypeStruct((BC, 16), jnp.float32),
        scratch_types=[
            pltpu.VMEM((rows_per_tile, L), jnp.float32),
            pltpu.VMEM((rows_per_tile, 16), jnp.float32),
        ],
        compiler_params=pltpu.CompilerParams(needs_layout_passes=False),
    )
    def sc_kernel(corr_hbm, out_hbm, rows_v, tops_v):
        wid = lax.axis_index("s") * info.num_cores + lax.axis_index("c")
        base = wid * rows_per_tile
        pltpu.sync_copy(corr_hbm.at[pl.ds(base, rows_per_tile)], rows_v)
        for r in range(rows_per_tile):
            def chunk(j, t_asc):
                v = jnp.abs(rows_v[r, pl.ds(j * 16, 16)])
                vmax = jnp.max(v)
                tmin = jnp.min(t_asc)

                def merge(_):
                    vd, _unused = plsc.sort_key_val(v, v, descending=True)
                    u = jnp.maximum(t_asc, vd)
                    ts, _unused2 = plsc.sort_key_val(u, u)
                    return ts

                return lax.cond(vmax > tmin, merge, lambda _: t_asc, 0)

            t16 = lax.fori_loop(0, nchunks, chunk,
                                jnp.full((16,), -1.0, jnp.float32))
            tops_v[r] = t16
        pltpu.sync_copy(tops_v, out_hbm.at[pl.ds(base, rows_per_tile)])

    return sc_kernel(corr_t)


def _encoder_tail_kernel(corr_ref, seas_ref, tops_ref, wk_ref, b_ref,
                         scale_ref, bias_ref, out_ref):
    BC, L = corr_ref.shape
    C = out_ref.shape[1]
    B = BC // C
    hi = jax.lax.Precision.HIGHEST

    c = corr_ref[...]
    mag = jnp.abs(c)
    t = tops_ref[:, 0:1]
    iota = jax.lax.broadcasted_iota(jnp.int32, (BC, L), 1)

    gt = mag > t
    ng = jnp.sum(gt.astype(jnp.int32), axis=1)
    need = TOPK - ng
    eq = mag == t
    cur = jnp.where(eq, iota, L)
    sel = gt
    for k in range(_NTIES):
        imin = jnp.min(cur, axis=1)
        pick = (iota == imin[:, None]) & (need > k)[:, None]
        sel = sel | pick
        cur = jnp.where(iota == imin[:, None], L, cur)

    masked = jnp.where(sel, c, 0.0)
    ff = jax.lax.dot_general(
        wk_ref[...], masked, (((0,), (0,)), ((), ())),
        preferred_element_type=jnp.float32, precision=hi)
    x = seas_ref[...] + ff + b_ref[...]

    xr = x.reshape(B, C, L)
    mean = jnp.mean(xr, axis=1, keepdims=True)
    xc = xr - mean
    var = jnp.mean(xc * xc, axis=1, keepdims=True)
    normed = xc * jax.lax.rsqrt(var + EPS)
    o = normed.reshape(BC, L) * scale_ref[...] + bias_ref[...]
    for i in range(B):
        out_ref[pl.ds(i * L, L), :] = o[i * C:(i + 1) * C, :].T


def kernel(seasonal, trend, W, b, ln_scale, ln_bias):
    B, L, C = seasonal.shape
    D = W.shape[1]
    X = jnp.fft.rfft(seasonal, axis=1)
    P = X * jnp.conj(X)
    corr = jnp.fft.irfft(P, n=L, axis=1)

    corr_t = jax.lax.optimization_barrier(
        jnp.transpose(corr, (0, 2, 1)).reshape(B * C, L))
    seas_t = jnp.transpose(seasonal, (0, 2, 1)).reshape(B * C, L)

    tops = _sc_top16(corr_t)

    wk = jnp.kron(jnp.eye(B, dtype=jnp.float32), W)
    bt = jnp.tile(b, B).reshape(B * D, 1)
    st = jnp.tile(ln_scale, B).reshape(B * D, 1)
    bst = jnp.tile(ln_bias, B).reshape(B * D, 1)

    out2 = pl.pallas_call(
        _encoder_tail_kernel,
        out_shape=jax.ShapeDtypeStruct((B * L, D), jnp.float32),
    )(corr_t, seas_t, tops, wk, bt, st, bst)
    return (out2.reshape(B, L, D), trend)
